# Initial kernel scaffold; baseline (speedup 1.0000x reference)
import functools

import jax
import jax.numpy as jnp
from jax import lax
from jax.experimental import pallas as pl
from jax.experimental.pallas import tpu as pltpu
from jax.experimental.pallas import tpu_sc as plsc

N = 10000
E = 320000
D = 128
EA = 16
NC = 2
NS = 16
NW = NC * NS
C = 128
NCHUNK = E // C
K0 = NCHUNK // NW
REM = NCHUNK - K0 * NW
RPT = N // NS


def _sc_aggregate():
    mesh = plsc.VectorSubcoreMesh(
        core_axis_name="c", subcore_axis_name="s",
        num_cores=NC, num_subcores=NS)

    @functools.partial(
        pl.kernel,
        out_type=(
            jax.ShapeDtypeStruct((NC * N, D), jnp.float32),
            jax.ShapeDtypeStruct((NC * N, EA), jnp.float32),
            jax.ShapeDtypeStruct((NC * N, EA), jnp.float32),
        ),
        mesh=mesh,
        scratch_types=[
            pltpu.VMEM_SHARED((N, D), jnp.float32),
            pltpu.VMEM_SHARED((N, EA), jnp.float32),
            pltpu.VMEM_SHARED((N, EA), jnp.float32),
            pltpu.VMEM((C,), jnp.int32),
            pltpu.VMEM((C,), jnp.int32),
            pltpu.VMEM((C, D), jnp.float32),
            pltpu.VMEM((C, EA), jnp.float32),
            pltpu.VMEM((C, EA), jnp.float32),
            pltpu.SemaphoreType.DMA,
        ],
    )
    def sc(x_hbm, src_hbm, dst_hbm, ea_hbm, z128_hbm, z16_hbm,
           px_hbm, pe_hbm, pd_hbm,
           acc_x, acc_e, acc_d, srcv, dstv, xrows, eabuf, onesb, sem):
        cid = lax.axis_index("c")
        sid = lax.axis_index("s")
        wid = sid * NC + cid
        r0 = sid * RPT

        @pl.loop(0, C)
        def _(r):
            onesb[r, :] = jnp.ones((EA,), jnp.float32)

        pltpu.sync_copy(z128_hbm.at[pl.ds(r0, RPT)], acc_x.at[pl.ds(r0, RPT)])
        pltpu.sync_copy(z16_hbm.at[pl.ds(r0, RPT)], acc_e.at[pl.ds(r0, RPT)])
        pltpu.sync_copy(z16_hbm.at[pl.ds(r0, RPT)], acc_d.at[pl.ds(r0, RPT)])
        plsc.subcore_barrier()

        def do_chunk(c):
            base = c * C
            pltpu.sync_copy(src_hbm.at[pl.ds(base, C)], srcv)
            pltpu.sync_copy(dst_hbm.at[pl.ds(base, C)], dstv)
            pltpu.async_copy(x_hbm.at[srcv], xrows, sem).wait()
            pltpu.sync_copy(xrows, acc_x.at[dstv], add=True)
            pltpu.sync_copy(ea_hbm.at[pl.ds(base, C)], eabuf)
            pltpu.sync_copy(eabuf, acc_e.at[dstv], add=True)
            pltpu.sync_copy(onesb, acc_d.at[dstv], add=True)

        @pl.loop(0, K0)
        def _(k):
            do_chunk(wid + NW * k)

        @pl.when(wid < REM)
        def _():
            do_chunk(K0 * NW + wid)

        plsc.subcore_barrier()
        off = cid * N + r0
        pltpu.sync_copy(acc_x.at[pl.ds(r0, RPT)], px_hbm.at[pl.ds(off, RPT)])
        pltpu.sync_copy(acc_e.at[pl.ds(r0, RPT)], pe_hbm.at[pl.ds(off, RPT)])
        pltpu.sync_copy(acc_d.at[pl.ds(r0, RPT)], pd_hbm.at[pl.ds(off, RPT)])

    return sc


def _tc_body(px, pe, pd, x, wx, wself, we, bx, bself, be, out):
    gx = px[0] + px[1]
    ga = pe[0] + pe[1]
    deg = pd[0, :, 0:1] + pd[1, :, 0:1]
    summed = jnp.dot(gx, wx[...], preferred_element_type=jnp.float32)
    summed = summed + jnp.dot(ga, we[...], preferred_element_type=jnp.float32)
    summed = summed + deg * (bx[...] + be[...])
    agg = summed / jnp.maximum(deg, 1.0)
    out[...] = agg + jnp.dot(x[...], wself[...],
                             preferred_element_type=jnp.float32) + bself[...]


def kernel(x, edge_index, edge_attr, W_x, b_x, W_self, b_self, W_e, b_e):
    src = edge_index[0].astype(jnp.int32)
    dst = edge_index[1].astype(jnp.int32)
    z128 = jnp.zeros((N, D), jnp.float32)
    z16 = jnp.zeros((N, EA), jnp.float32)

    px, pe, pd = _sc_aggregate()(x, src, dst, edge_attr, z128, z16)

    out = pl.pallas_call(
        _tc_body,
        out_shape=jax.ShapeDtypeStruct((N, D), jnp.float32),
    )(px.reshape(NC, N, D), pe.reshape(NC, N, EA), pd.reshape(NC, N, EA),
      x, W_x, W_self, W_e,
      b_x.reshape(1, D), b_self.reshape(1, D), b_e.reshape(1, D))
    return out


# speedup vs baseline: 2.7583x; 2.7583x over previous
import functools

import jax
import jax.numpy as jnp
from jax import lax
from jax.experimental import pallas as pl
from jax.experimental.pallas import tpu as pltpu
from jax.experimental.pallas import tpu_sc as plsc

N = 10000
E = 320000
D = 128
EA = 16
NC = 2
NS = 16
C = 64
NCHUNK = E // C
K0 = NCHUNK // NS
REM = NCHUNK - K0 * NS
RPT = 624
TAIL = N - NS * RPT
L = 16
ER = C * EA // D


def _zero_vmem(ref, rows, width):
    z = jnp.zeros((L,), jnp.float32)

    @pl.loop(0, rows)
    def _(r):
        for cc in range(width // L):
            ref[r, pl.ds(cc * L, L)] = z


def _striped(copy, r0, total, chunk):
    done = 0
    while done < total:
        n = min(chunk, total - done)
        copy(r0 + done, n)
        done += n


def _sc_aggregate():
    mesh = plsc.VectorSubcoreMesh(
        core_axis_name="c", subcore_axis_name="s",
        num_cores=NC, num_subcores=NS)

    @functools.partial(
        pl.kernel,
        out_type=(
            jax.ShapeDtypeStruct((N, D), jnp.float32),
            jax.ShapeDtypeStruct((N, D), jnp.float32),
        ),
        mesh=mesh,
        scratch_types=[
            pltpu.VMEM_SHARED((N, D), jnp.float32),
            pltpu.VMEM((C,), jnp.int32),
            pltpu.VMEM((C,), jnp.int32),
            pltpu.VMEM((C, D), jnp.float32),
            pltpu.VMEM((ER, D), jnp.float32),
            pltpu.VMEM((C, D), jnp.float32),
            pltpu.SemaphoreType.DMA,
        ],
    )
    def sc(x_hbm, src_hbm, dst_hbm, ea8_hbm,
           px_hbm, pm_hbm,
           acc, srcv, dstv, xrows, eabuf, mbuf, sem):
        cid = lax.axis_index("c")
        sid = lax.axis_index("s")
        r0 = sid * RPT
        last = sid == NS - 1

        _zero_vmem(xrows, C, D)

        def zero(lo, n):
            pltpu.sync_copy(xrows.at[pl.ds(0, n)], acc.at[pl.ds(lo, n)])

        _striped(zero, r0, RPT, C)

        @pl.when(last)
        def _():
            zero(N - TAIL, TAIL)

        _zero_vmem(mbuf, C, D)
        ones = jnp.ones((L,), jnp.float32)

        @pl.loop(0, C)
        def _(r):
            mbuf[r, pl.ds(EA, L)] = ones

        plsc.subcore_barrier()

        def do_chunk_x(c):
            base = c * C
            pltpu.sync_copy(src_hbm.at[pl.ds(base, C)], srcv)
            pltpu.sync_copy(dst_hbm.at[pl.ds(base, C)], dstv)
            pltpu.async_copy(x_hbm.at[srcv], xrows, sem).wait()
            pltpu.sync_copy(xrows, acc.at[dstv], add=True)

        def do_chunk_m(c):
            base = c * C
            pltpu.sync_copy(dst_hbm.at[pl.ds(base, C)], dstv)
            pltpu.sync_copy(ea8_hbm.at[pl.ds(c * ER, ER)], eabuf)
            for e in range(C):
                mbuf[e, pl.ds(0, EA)] = eabuf[e // (D // EA),
                                              pl.ds((e % (D // EA)) * EA, EA)]
            pltpu.sync_copy(mbuf, acc.at[dstv], add=True)

        @pl.when(cid == 0)
        def _():
            @pl.loop(0, K0)
            def _(k):
                do_chunk_x(sid + NS * k)

            @pl.when(sid < REM)
            def _():
                do_chunk_x(K0 * NS + sid)

        @pl.when(cid == 1)
        def _():
            @pl.loop(0, K0)
            def _(k):
                do_chunk_m(sid + NS * k)

            @pl.when(sid < REM)
            def _():
                do_chunk_m(K0 * NS + sid)

        plsc.subcore_barrier()

        def drain(out_hbm):
            def d(lo, n):
                pltpu.sync_copy(acc.at[pl.ds(lo, n)], xrows.at[pl.ds(0, n)])
                pltpu.sync_copy(xrows.at[pl.ds(0, n)], out_hbm.at[pl.ds(lo, n)])

            _striped(d, r0, RPT, C)

            @pl.when(last)
            def _():
                d(N - TAIL, TAIL)

        @pl.when(cid == 0)
        def _():
            drain(px_hbm)

        @pl.when(cid == 1)
        def _():
            drain(pm_hbm)

    return sc


def _tc_body(px, pm, x, wx, wself, we, bx, bself, be, out):
    gx = px[...]
    ga = pm[:, 0:EA]
    deg = pm[:, EA:EA + 1]
    summed = jnp.dot(gx, wx[...], preferred_element_type=jnp.float32)
    summed = summed + jnp.dot(ga, we[...], preferred_element_type=jnp.float32)
    summed = summed + deg * (bx[...] + be[...])
    agg = summed / jnp.maximum(deg, 1.0)
    out[...] = agg + jnp.dot(x[...], wself[...],
                             preferred_element_type=jnp.float32) + bself[...]


def kernel(x, edge_index, edge_attr, W_x, b_x, W_self, b_self, W_e, b_e):
    src = edge_index[0].astype(jnp.int32)
    dst = edge_index[1].astype(jnp.int32)
    ea8 = edge_attr.reshape(E * EA // D, D)

    px, pm = _sc_aggregate()(x, src, dst, ea8)

    out = pl.pallas_call(
        _tc_body,
        out_shape=jax.ShapeDtypeStruct((N, D), jnp.float32),
    )(px, pm, x, W_x, W_self, W_e,
      b_x.reshape(1, D), b_self.reshape(1, D), b_e.reshape(1, D))
    return out


# speedup vs baseline: 5.0947x; 1.8470x over previous
import functools

import jax
import jax.numpy as jnp
from jax import lax
from jax.experimental import pallas as pl
from jax.experimental.pallas import tpu as pltpu
from jax.experimental.pallas import tpu_sc as plsc

N = 10000
E = 320000
D = 128
EA = 16
NC = 2
NS = 16
C = 64
SUP = 8
NCHUNK = E // C
K0 = NCHUNK // NS
NSUP = K0 // SUP
REM = NCHUNK - K0 * NS
RPT = 624
TAIL = N - NS * RPT
L = 16
ER = C * EA // D
EPC = D // EA


def _zero_vmem(ref, rows, width):
    z = jnp.zeros((L,), jnp.float32)

    @pl.loop(0, rows)
    def _(r):
        for cc in range(width // L):
            ref[r, pl.ds(cc * L, L)] = z


def _striped(copy, r0, total, chunk):
    done = 0
    while done < total:
        n = min(chunk, total - done)
        copy(r0 + done, n)
        done += n


def _sc_aggregate():
    mesh = plsc.VectorSubcoreMesh(
        core_axis_name="c", subcore_axis_name="s",
        num_cores=NC, num_subcores=NS)

    @functools.partial(
        pl.kernel,
        out_type=(
            jax.ShapeDtypeStruct((N, D), jnp.float32),
            jax.ShapeDtypeStruct((N, D), jnp.float32),
        ),
        mesh=mesh,
        scratch_types=[
            pltpu.VMEM_SHARED((N, D), jnp.float32),
            pltpu.VMEM((C, D), jnp.float32),
            pltpu.VMEM((C, D), jnp.float32),
            pltpu.VMEM((SUP, C), jnp.int32),
            pltpu.VMEM((SUP, C), jnp.int32),
            pltpu.VMEM((SUP * ER, D), jnp.float32),
            pltpu.SemaphoreType.DMA,
            pltpu.SemaphoreType.DMA,
        ],
    )
    def sc(x_hbm, ei2_hbm, ea8_hbm,
           px_hbm, pm_hbm,
           acc, bufA, bufB, srcall, dstall, eaall, semA, semB):
        cid = lax.axis_index("c")
        sid = lax.axis_index("s")
        r0 = sid * RPT
        last = sid == NS - 1
        bufs = (bufA, bufB)
        sems = (semA, semB)
        cbase = sid * K0

        _zero_vmem(bufA, C, D)

        def zero(lo, n):
            pltpu.sync_copy(bufA.at[pl.ds(0, n)], acc.at[pl.ds(lo, n)])

        _striped(zero, r0, RPT, C)

        @pl.when(last)
        def _():
            zero(N - TAIL, TAIL)

        _zero_vmem(bufB, C, D)
        ones = jnp.ones((L,), jnp.float32)

        @pl.loop(0, C)
        def _(r):
            bufA[r, pl.ds(EA, L)] = ones
            bufB[r, pl.ds(EA, L)] = ones

        plsc.subcore_barrier()

        @pl.when(cid == 0)
        def _():
            @pl.loop(0, NSUP)
            def _(s):
                row = cbase + s * SUP
                pltpu.sync_copy(ei2_hbm.at[pl.ds(row, SUP)], srcall)
                pltpu.sync_copy(ei2_hbm.at[pl.ds(NCHUNK + row, SUP)], dstall)
                descs = [None, None]
                descs[0] = pltpu.async_copy(
                    x_hbm.at[srcall.at[0]], bufs[0], sems[0])
                for j in range(SUP):
                    p = j % 2
                    if j + 1 < SUP:
                        descs[1 - p] = pltpu.async_copy(
                            x_hbm.at[srcall.at[j + 1]], bufs[1 - p],
                            sems[1 - p])
                    descs[p].wait()
                    pltpu.sync_copy(bufs[p], acc.at[dstall.at[j]], add=True)

            @pl.when(sid < REM)
            def _():
                row = K0 * NS
                pltpu.sync_copy(ei2_hbm.at[pl.ds(row, SUP)], srcall)
                pltpu.sync_copy(ei2_hbm.at[pl.ds(NCHUNK + row, SUP)], dstall)
                pltpu.async_copy(x_hbm.at[srcall.at[sid]], bufA, semA).wait()
                pltpu.sync_copy(bufA, acc.at[dstall.at[sid]], add=True)

        @pl.when(cid == 1)
        def _():
            def build(buf, ea_lo):
                for e in range(C):
                    buf[e, pl.ds(0, EA)] = eaall[ea_lo + e // EPC,
                                                 pl.ds((e % EPC) * EA, EA)]

            @pl.loop(0, NSUP)
            def _(s):
                row = cbase + s * SUP
                pltpu.sync_copy(ei2_hbm.at[pl.ds(NCHUNK + row, SUP)], dstall)
                pltpu.sync_copy(ea8_hbm.at[pl.ds(row * ER, SUP * ER)], eaall)
                descs = [None, None]
                for j in range(SUP):
                    p = j % 2
                    if descs[p] is not None:
                        descs[p].wait()
                    build(bufs[p], j * ER)
                    descs[p] = pltpu.async_copy(
                        bufs[p], acc.at[dstall.at[j]], sems[p], add=True)
                descs[0].wait()
                descs[1].wait()

            @pl.when(sid < REM)
            def _():
                row = K0 * NS
                pltpu.sync_copy(ei2_hbm.at[pl.ds(NCHUNK + row, SUP)], dstall)
                pltpu.sync_copy(ea8_hbm.at[pl.ds((row + sid) * ER, ER)],
                                eaall.at[pl.ds(0, ER)])
                build(bufA, 0)
                pltpu.sync_copy(bufA, acc.at[dstall.at[sid]], add=True)

        plsc.subcore_barrier()

        def drain(out_hbm):
            def d(lo, n):
                pltpu.sync_copy(acc.at[pl.ds(lo, n)], bufA.at[pl.ds(0, n)])
                pltpu.sync_copy(bufA.at[pl.ds(0, n)], out_hbm.at[pl.ds(lo, n)])

            _striped(d, r0, RPT, C)

            @pl.when(last)
            def _():
                d(N - TAIL, TAIL)

        @pl.when(cid == 0)
        def _():
            drain(px_hbm)

        @pl.when(cid == 1)
        def _():
            drain(pm_hbm)

    return sc


def _tc_body(px, pm, x, wx, wself, we, bx, bself, be, out):
    gx = px[...]
    ga = pm[:, 0:EA]
    deg = pm[:, EA:EA + 1]
    summed = jnp.dot(gx, wx[...], preferred_element_type=jnp.float32)
    summed = summed + jnp.dot(ga, we[...], preferred_element_type=jnp.float32)
    summed = summed + deg * (bx[...] + be[...])
    agg = summed / jnp.maximum(deg, 1.0)
    out[...] = agg + jnp.dot(x[...], wself[...],
                             preferred_element_type=jnp.float32) + bself[...]


def kernel(x, edge_index, edge_attr, W_x, b_x, W_self, b_self, W_e, b_e):
    ei2 = edge_index.astype(jnp.int32).reshape(2 * NCHUNK, C)
    ea8 = edge_attr.reshape(E * EA // D, D)

    px, pm = _sc_aggregate()(x, ei2, ea8)

    out = pl.pallas_call(
        _tc_body,
        out_shape=jax.ShapeDtypeStruct((N, D), jnp.float32),
    )(px, pm, x, W_x, W_self, W_e,
      b_x.reshape(1, D), b_self.reshape(1, D), b_e.reshape(1, D))
    return out


# speedup vs baseline: 5.5680x; 1.0929x over previous
import functools

import jax
import jax.numpy as jnp
from jax import lax
from jax.experimental import pallas as pl
from jax.experimental.pallas import tpu as pltpu
from jax.experimental.pallas import tpu_sc as plsc

N = 10000
E = 320000
D = 128
EA = 16
NC = 2
NS = 16
C = 64
SUP = 8
NCHUNK = E // C
K0 = NCHUNK // NS
NSUP = K0 // SUP
REM = NCHUNK - K0 * NS
RPT = 624
TAIL = N - NS * RPT
L = 16
ER = C * EA // D
EPC = D // EA


def _zero_vmem(ref, rows, width):
    z = jnp.zeros((L,), jnp.float32)

    @pl.loop(0, rows)
    def _(r):
        for cc in range(width // L):
            ref[r, pl.ds(cc * L, L)] = z


def _striped(copy, r0, total, chunk):
    done = 0
    while done < total:
        n = min(chunk, total - done)
        copy(r0 + done, n)
        done += n


def _sc_aggregate():
    mesh = plsc.VectorSubcoreMesh(
        core_axis_name="c", subcore_axis_name="s",
        num_cores=NC, num_subcores=NS)

    @functools.partial(
        pl.kernel,
        out_type=(
            jax.ShapeDtypeStruct((N, D), jnp.float32),
            jax.ShapeDtypeStruct((N, D), jnp.float32),
        ),
        mesh=mesh,
        scratch_types=[
            pltpu.VMEM_SHARED((N, D), jnp.float32),
            pltpu.VMEM((C, D), jnp.float32),
            pltpu.VMEM((C, D), jnp.float32),
            pltpu.VMEM((C, D), jnp.float32),
            pltpu.VMEM((SUP, C), jnp.int32),
            pltpu.VMEM((SUP, C), jnp.int32),
            pltpu.VMEM((SUP * ER // 2, D), jnp.float32),
            pltpu.SemaphoreType.DMA,
            pltpu.SemaphoreType.DMA,
            pltpu.SemaphoreType.DMA,
            pltpu.SemaphoreType.DMA,
            pltpu.SemaphoreType.DMA,
            pltpu.SemaphoreType.DMA,
        ],
    )
    def sc(x_hbm, ei2_hbm, ea8_hbm,
           px_hbm, pm_hbm,
           acc, bufA, bufB, bufC, srcall, dstall, eaall,
           semA, semB, semC, semSA, semSB, semSC):
        cid = lax.axis_index("c")
        sid = lax.axis_index("s")
        r0 = sid * RPT
        last = sid == NS - 1
        bufs = (bufA, bufB, bufC)
        gsems = (semA, semB, semC)
        ssems = (semSA, semSB, semSC)
        cbase = sid * K0

        _zero_vmem(bufA, C, D)

        def zero(lo, n):
            pltpu.sync_copy(bufA.at[pl.ds(0, n)], acc.at[pl.ds(lo, n)])

        _striped(zero, r0, RPT, C)

        @pl.when(last)
        def _():
            zero(N - TAIL, TAIL)

        _zero_vmem(bufB, C, D)
        _zero_vmem(bufC, C, D)
        ones = jnp.ones((L,), jnp.float32)

        @pl.loop(0, C)
        def _(r):
            bufA[r, pl.ds(EA, L)] = ones
            bufB[r, pl.ds(EA, L)] = ones
            bufC[r, pl.ds(EA, L)] = ones

        plsc.subcore_barrier()

        @pl.when(cid == 0)
        def _():
            @pl.loop(0, NSUP)
            def _(s):
                row = cbase + s * SUP
                pltpu.sync_copy(ei2_hbm.at[pl.ds(row, SUP)], srcall)
                pltpu.sync_copy(ei2_hbm.at[pl.ds(NCHUNK + row, SUP)], dstall)
                gd = [None] * SUP
                sd = [None] * SUP
                gd[0] = pltpu.async_copy(
                    x_hbm.at[srcall.at[0]], bufs[0], gsems[0])
                gd[1] = pltpu.async_copy(
                    x_hbm.at[srcall.at[1]], bufs[1], gsems[1])
                for j in range(SUP):
                    p = j % 3
                    gd[j].wait()
                    if j + 2 < SUP:
                        if j >= 1:
                            sd[j - 1].wait()
                        gd[j + 2] = pltpu.async_copy(
                            x_hbm.at[srcall.at[j + 2]], bufs[(j + 2) % 3],
                            gsems[(j + 2) % 3])
                    sd[j] = pltpu.async_copy(
                        bufs[p], acc.at[dstall.at[j]], ssems[p], add=True)
                sd[SUP - 3].wait()
                sd[SUP - 2].wait()
                sd[SUP - 1].wait()

            @pl.when(sid < REM)
            def _():
                row = K0 * NS
                pltpu.sync_copy(ei2_hbm.at[pl.ds(row, SUP)], srcall)
                pltpu.sync_copy(ei2_hbm.at[pl.ds(NCHUNK + row, SUP)], dstall)
                pltpu.async_copy(x_hbm.at[srcall.at[sid]], bufA, semA).wait()
                pltpu.sync_copy(bufA, acc.at[dstall.at[sid]], add=True)

        @pl.when(cid == 1)
        def _():
            def build(buf, ea_lo):
                for e in range(C):
                    buf[e, pl.ds(0, EA)] = eaall[ea_lo + e // EPC,
                                                 pl.ds((e % EPC) * EA, EA)]

            @pl.loop(0, NSUP)
            def _(s):
                row = cbase + s * SUP
                pltpu.sync_copy(ei2_hbm.at[pl.ds(NCHUNK + row, SUP)], dstall)
                sd = [None] * SUP
                for j in range(SUP):
                    p = j % 3
                    if j % (SUP // 2) == 0:
                        pltpu.sync_copy(
                            ea8_hbm.at[pl.ds((row + j) * ER, SUP * ER // 2)],
                            eaall)
                    if j >= 3:
                        sd[j - 3].wait()
                    build(bufs[p], (j % (SUP // 2)) * ER)
                    sd[j] = pltpu.async_copy(
                        bufs[p], acc.at[dstall.at[j]], ssems[p], add=True)
                sd[SUP - 3].wait()
                sd[SUP - 2].wait()
                sd[SUP - 1].wait()

            @pl.when(sid < REM)
            def _():
                row = K0 * NS
                pltpu.sync_copy(ei2_hbm.at[pl.ds(NCHUNK + row, SUP)], dstall)
                pltpu.sync_copy(ea8_hbm.at[pl.ds((row + sid) * ER, ER)],
                                eaall.at[pl.ds(0, ER)])
                build(bufA, 0)
                pltpu.sync_copy(bufA, acc.at[dstall.at[sid]], add=True)

        plsc.subcore_barrier()

        def drain(out_hbm):
            def d(lo, n):
                pltpu.sync_copy(acc.at[pl.ds(lo, n)], bufA.at[pl.ds(0, n)])
                pltpu.sync_copy(bufA.at[pl.ds(0, n)], out_hbm.at[pl.ds(lo, n)])

            _striped(d, r0, RPT, C)

            @pl.when(last)
            def _():
                d(N - TAIL, TAIL)

        @pl.when(cid == 0)
        def _():
            drain(px_hbm)

        @pl.when(cid == 1)
        def _():
            drain(pm_hbm)

    return sc


def _tc_body(px, pm, x, wx, wself, we, bx, bself, be, out):
    gx = px[...]
    ga = pm[:, 0:EA]
    deg = pm[:, EA:EA + 1]
    summed = jnp.dot(gx, wx[...], preferred_element_type=jnp.float32)
    summed = summed + jnp.dot(ga, we[...], preferred_element_type=jnp.float32)
    summed = summed + deg * (bx[...] + be[...])
    agg = summed / jnp.maximum(deg, 1.0)
    out[...] = agg + jnp.dot(x[...], wself[...],
                             preferred_element_type=jnp.float32) + bself[...]


def kernel(x, edge_index, edge_attr, W_x, b_x, W_self, b_self, W_e, b_e):
    ei2 = edge_index.astype(jnp.int32).reshape(2 * NCHUNK, C)
    ea8 = edge_attr.reshape(E * EA // D, D)

    px, pm = _sc_aggregate()(x, ei2, ea8)

    out = pl.pallas_call(
        _tc_body,
        out_shape=jax.ShapeDtypeStruct((N, D), jnp.float32),
    )(px, pm, x, W_x, W_self, W_e,
      b_x.reshape(1, D), b_self.reshape(1, D), b_e.reshape(1, D))
    return out


# speedup vs baseline: 5.5976x; 1.0053x over previous
import functools

import jax
import jax.numpy as jnp
from jax import lax
from jax.experimental import pallas as pl
from jax.experimental.pallas import tpu as pltpu
from jax.experimental.pallas import tpu_sc as plsc

N = 10000
E = 320000
D = 128
EA = 16
NC = 2
NS = 16
C = 64
SUP = 8
NCHUNK = E // C
K0 = NCHUNK // NS
NSUP = K0 // SUP
REM = NCHUNK - K0 * NS
RPT = 624
TAIL = N - NS * RPT
L = 16
ER = C * EA // D
EPC = D // EA


def _zero_vmem(ref, rows, width):
    z = jnp.zeros((L,), jnp.float32)

    @pl.loop(0, rows)
    def _(r):
        for cc in range(width // L):
            ref[r, pl.ds(cc * L, L)] = z


def _striped(copy, r0, total, chunk):
    done = 0
    while done < total:
        n = min(chunk, total - done)
        copy(r0 + done, n)
        done += n


def _sc_aggregate():
    mesh = plsc.VectorSubcoreMesh(
        core_axis_name="c", subcore_axis_name="s",
        num_cores=NC, num_subcores=NS)

    @functools.partial(
        pl.kernel,
        out_type=(
            jax.ShapeDtypeStruct((N, D), jnp.float32),
            jax.ShapeDtypeStruct((N, D), jnp.float32),
        ),
        mesh=mesh,
        scratch_types=[
            pltpu.VMEM_SHARED((N, D), jnp.float32),
            pltpu.VMEM((C, D), jnp.float32),
            pltpu.VMEM((C, D), jnp.float32),
            pltpu.VMEM((C, D), jnp.float32),
            pltpu.VMEM((SUP * C,), jnp.int32),
            pltpu.VMEM((SUP * C,), jnp.int32),
            pltpu.VMEM((SUP * ER // 2, D), jnp.float32),
            pltpu.SemaphoreType.DMA,
            pltpu.SemaphoreType.DMA,
            pltpu.SemaphoreType.DMA,
            pltpu.SemaphoreType.DMA,
            pltpu.SemaphoreType.DMA,
            pltpu.SemaphoreType.DMA,
        ],
    )
    def sc(x_hbm, ei_hbm, ea8_hbm,
           px_hbm, pm_hbm,
           acc, bufA, bufB, bufC, srcall, dstall, eaall,
           semA, semB, semC, semSA, semSB, semSC):
        cid = lax.axis_index("c")
        sid = lax.axis_index("s")
        r0 = sid * RPT
        last = sid == NS - 1
        bufs = (bufA, bufB, bufC)
        gsems = (semA, semB, semC)
        ssems = (semSA, semSB, semSC)
        cbase = sid * K0

        _zero_vmem(bufA, C, D)

        def zero(lo, n):
            pltpu.sync_copy(bufA.at[pl.ds(0, n)], acc.at[pl.ds(lo, n)])

        _striped(zero, r0, RPT, C)

        @pl.when(last)
        def _():
            zero(N - TAIL, TAIL)

        _zero_vmem(bufB, C, D)
        _zero_vmem(bufC, C, D)
        ones = jnp.ones((L,), jnp.float32)

        @pl.loop(0, C)
        def _(r):
            bufA[r, pl.ds(EA, L)] = ones
            bufB[r, pl.ds(EA, L)] = ones
            bufC[r, pl.ds(EA, L)] = ones

        plsc.subcore_barrier()

        @pl.when(cid == 0)
        def _():
            @pl.loop(0, NSUP)
            def _(s):
                base = (cbase + s * SUP) * C
                pltpu.sync_copy(ei_hbm.at[0, pl.ds(base, SUP * C)], srcall)
                pltpu.sync_copy(ei_hbm.at[1, pl.ds(base, SUP * C)], dstall)
                gd = [None] * SUP
                sd = [None] * SUP
                gd[0] = pltpu.async_copy(
                    x_hbm.at[srcall.at[pl.ds(0, C)]], bufs[0], gsems[0])
                gd[1] = pltpu.async_copy(
                    x_hbm.at[srcall.at[pl.ds(C, C)]], bufs[1], gsems[1])
                for j in range(SUP):
                    p = j % 3
                    gd[j].wait()
                    if j + 2 < SUP:
                        if j >= 1:
                            sd[j - 1].wait()
                        gd[j + 2] = pltpu.async_copy(
                            x_hbm.at[srcall.at[pl.ds((j + 2) * C, C)]],
                            bufs[(j + 2) % 3], gsems[(j + 2) % 3])
                    sd[j] = pltpu.async_copy(
                        bufs[p], acc.at[dstall.at[pl.ds(j * C, C)]],
                        ssems[p], add=True)
                sd[SUP - 3].wait()
                sd[SUP - 2].wait()
                sd[SUP - 1].wait()

            @pl.when(sid < REM)
            def _():
                base = K0 * NS * C
                pltpu.sync_copy(ei_hbm.at[0, pl.ds(base, SUP * C)], srcall)
                pltpu.sync_copy(ei_hbm.at[1, pl.ds(base, SUP * C)], dstall)
                pltpu.async_copy(
                    x_hbm.at[srcall.at[pl.ds(sid * C, C)]], bufA, semA).wait()
                pltpu.sync_copy(
                    bufA, acc.at[dstall.at[pl.ds(sid * C, C)]], add=True)

        @pl.when(cid == 1)
        def _():
            def build(buf, ea_lo):
                for e in range(C):
                    buf[e, pl.ds(0, EA)] = eaall[ea_lo + e // EPC,
                                                 pl.ds((e % EPC) * EA, EA)]

            @pl.loop(0, NSUP)
            def _(s):
                row = cbase + s * SUP
                base = row * C
                pltpu.sync_copy(ei_hbm.at[1, pl.ds(base, SUP * C)], dstall)
                sd = [None] * SUP
                for j in range(SUP):
                    p = j % 3
                    if j % (SUP // 2) == 0:
                        pltpu.sync_copy(
                            ea8_hbm.at[pl.ds((row + j) * ER, SUP * ER // 2)],
                            eaall)
                    if j >= 3:
                        sd[j - 3].wait()
                    build(bufs[p], (j % (SUP // 2)) * ER)
                    sd[j] = pltpu.async_copy(
                        bufs[p], acc.at[dstall.at[pl.ds(j * C, C)]],
                        ssems[p], add=True)
                sd[SUP - 3].wait()
                sd[SUP - 2].wait()
                sd[SUP - 1].wait()

            @pl.when(sid < REM)
            def _():
                row = K0 * NS
                pltpu.sync_copy(ei_hbm.at[1, pl.ds(row * C, SUP * C)], dstall)
                pltpu.sync_copy(ea8_hbm.at[pl.ds((row + sid) * ER, ER)],
                                eaall.at[pl.ds(0, ER)])
                build(bufA, 0)
                pltpu.sync_copy(
                    bufA, acc.at[dstall.at[pl.ds(sid * C, C)]], add=True)

        plsc.subcore_barrier()

        def drain(out_hbm):
            def d(lo, n):
                pltpu.sync_copy(acc.at[pl.ds(lo, n)], bufA.at[pl.ds(0, n)])
                pltpu.sync_copy(bufA.at[pl.ds(0, n)], out_hbm.at[pl.ds(lo, n)])

            _striped(d, r0, RPT, C)

            @pl.when(last)
            def _():
                d(N - TAIL, TAIL)

        @pl.when(cid == 0)
        def _():
            drain(px_hbm)

        @pl.when(cid == 1)
        def _():
            drain(pm_hbm)

    return sc


def _tc_body(px, pm, x, wx, wself, we, bx, bself, be, out):
    gx = px[...]
    ga = pm[:, 0:EA]
    deg = pm[:, EA:EA + 1]
    summed = jnp.dot(gx, wx[...], preferred_element_type=jnp.float32)
    summed = summed + jnp.dot(ga, we[...], preferred_element_type=jnp.float32)
    summed = summed + deg * (bx[...] + be[...])
    agg = summed / jnp.maximum(deg, 1.0)
    out[...] = agg + jnp.dot(x[...], wself[...],
                             preferred_element_type=jnp.float32) + bself[...]


def kernel(x, edge_index, edge_attr, W_x, b_x, W_self, b_self, W_e, b_e):
    ei = edge_index.astype(jnp.int32)
    ea8 = edge_attr.reshape(E * EA // D, D)

    px, pm = _sc_aggregate()(x, ei, ea8)

    RB = 1000
    out = pl.pallas_call(
        _tc_body,
        grid=(N // RB,),
        in_specs=[
            pl.BlockSpec((RB, D), lambda i: (i, 0)),
            pl.BlockSpec((RB, D), lambda i: (i, 0)),
            pl.BlockSpec((RB, D), lambda i: (i, 0)),
            pl.BlockSpec((D, D), lambda i: (0, 0)),
            pl.BlockSpec((D, D), lambda i: (0, 0)),
            pl.BlockSpec((EA, D), lambda i: (0, 0)),
            pl.BlockSpec((1, D), lambda i: (0, 0)),
            pl.BlockSpec((1, D), lambda i: (0, 0)),
            pl.BlockSpec((1, D), lambda i: (0, 0)),
        ],
        out_specs=pl.BlockSpec((RB, D), lambda i: (i, 0)),
        out_shape=jax.ShapeDtypeStruct((N, D), jnp.float32),
    )(px, pm, x, W_x, W_self, W_e,
      b_x.reshape(1, D), b_self.reshape(1, D), b_e.reshape(1, D))
    return out


# speedup vs baseline: 6.1739x; 1.1029x over previous
import functools

import jax
import jax.numpy as jnp
from jax import lax
from jax.experimental import pallas as pl
from jax.experimental.pallas import tpu as pltpu
from jax.experimental.pallas import tpu_sc as plsc

N = 10000
E = 320000
D = 128
EA = 16
NC = 2
NS = 16
C = 64
SUP = 8
NCHUNK = E // C
K0 = NCHUNK // NS
NSUP = K0 // SUP
REM = NCHUNK - K0 * NS
RPT = 624
TAIL = N - NS * RPT
L = 16
ER = C * EA // D
EPC = D // EA


def _zero_vmem(ref, rows, width):
    z = jnp.zeros((L,), jnp.float32)

    @pl.loop(0, rows)
    def _(r):
        for cc in range(width // L):
            ref[r, pl.ds(cc * L, L)] = z


def _striped(copy, r0, total, chunk):
    done = 0
    while done < total:
        n = min(chunk, total - done)
        copy(r0 + done, n)
        done += n


def _sc_aggregate():
    mesh = plsc.VectorSubcoreMesh(
        core_axis_name="c", subcore_axis_name="s",
        num_cores=NC, num_subcores=NS)

    @functools.partial(
        pl.kernel,
        out_type=(
            jax.ShapeDtypeStruct((N, D), jnp.float32),
            jax.ShapeDtypeStruct((N, D), jnp.float32),
        ),
        mesh=mesh,
        scratch_types=[
            pltpu.VMEM_SHARED((N, D), jnp.float32),
            pltpu.VMEM((C, D), jnp.float32),
            pltpu.VMEM((C, D), jnp.float32),
            pltpu.VMEM((C, D), jnp.float32),
            pltpu.VMEM((SUP * C,), jnp.int32),
            pltpu.VMEM((SUP * C,), jnp.int32),
            pltpu.VMEM((SUP * C,), jnp.int32),
            pltpu.VMEM((SUP * C,), jnp.int32),
            pltpu.VMEM((SUP * ER // 2, D), jnp.float32),
            pltpu.SemaphoreType.DMA,
            pltpu.SemaphoreType.DMA,
            pltpu.SemaphoreType.DMA,
            pltpu.SemaphoreType.DMA,
            pltpu.SemaphoreType.DMA,
            pltpu.SemaphoreType.DMA,
            pltpu.SemaphoreType.DMA,
            pltpu.SemaphoreType.DMA,
        ],
    )
    def sc(x_hbm, ei_hbm, ea8_hbm,
           px_hbm, pm_hbm,
           acc, bufA, bufB, bufC, srcA, dstA, srcB, dstB, eaall,
           semA, semB, semC, semSA, semSB, semSC, semIA, semIB):
        cid = lax.axis_index("c")
        sid = lax.axis_index("s")
        r0 = sid * RPT
        last = sid == NS - 1
        bufs = (bufA, bufB, bufC)
        gsems = (semA, semB, semC)
        ssems = (semSA, semSB, semSC)
        cbase = sid * K0

        _zero_vmem(bufA, C, D)

        def zero(lo, n):
            pltpu.sync_copy(bufA.at[pl.ds(0, n)], acc.at[pl.ds(lo, n)])

        _striped(zero, r0, RPT, C)

        @pl.when(last)
        def _():
            zero(N - TAIL, TAIL)

        _zero_vmem(bufB, C, D)
        _zero_vmem(bufC, C, D)
        ones = jnp.ones((L,), jnp.float32)

        @pl.loop(0, C)
        def _(r):
            bufA[r, pl.ds(EA, L)] = ones
            bufB[r, pl.ds(EA, L)] = ones
            bufC[r, pl.ds(EA, L)] = ones

        plsc.subcore_barrier()

        @pl.when(cid == 0)
        def _():
            def prefetch(s, srcall, dstall, sem):
                b = (cbase + s * SUP) * C
                d1 = pltpu.async_copy(
                    ei_hbm.at[0, pl.ds(b, SUP * C)], srcall, sem)
                d2 = pltpu.async_copy(
                    ei_hbm.at[1, pl.ds(b, SUP * C)], dstall, sem)
                return d1, d2

            def run_super(srcall, dstall):
                gd = [None] * SUP
                sd = [None] * SUP
                gd[0] = pltpu.async_copy(
                    x_hbm.at[srcall.at[pl.ds(0, C)]], bufs[0], gsems[0])
                gd[1] = pltpu.async_copy(
                    x_hbm.at[srcall.at[pl.ds(C, C)]], bufs[1], gsems[1])
                for j in range(SUP):
                    p = j % 3
                    gd[j].wait()
                    if j + 2 < SUP:
                        if j >= 1:
                            sd[j - 1].wait()
                        gd[j + 2] = pltpu.async_copy(
                            x_hbm.at[srcall.at[pl.ds((j + 2) * C, C)]],
                            bufs[(j + 2) % 3], gsems[(j + 2) % 3])
                    sd[j] = pltpu.async_copy(
                        bufs[p], acc.at[dstall.at[pl.ds(j * C, C)]],
                        ssems[p], add=True)
                sd[SUP - 3].wait()
                sd[SUP - 2].wait()
                sd[SUP - 1].wait()

            b0 = cbase * C
            pltpu.sync_copy(ei_hbm.at[0, pl.ds(b0, SUP * C)], srcA)
            pltpu.sync_copy(ei_hbm.at[1, pl.ds(b0, SUP * C)], dstA)

            @pl.loop(0, (NSUP - 1) // 2)
            def _(t):
                pfB = prefetch(2 * t + 1, srcB, dstB, semIB)
                run_super(srcA, dstA)
                pfB[0].wait()
                pfB[1].wait()
                pfA = prefetch(2 * t + 2, srcA, dstA, semIA)
                run_super(srcB, dstB)
                pfA[0].wait()
                pfA[1].wait()

            run_super(srcA, dstA)

            @pl.when(sid < REM)
            def _():
                base = K0 * NS * C
                pltpu.sync_copy(ei_hbm.at[0, pl.ds(base, SUP * C)], srcA)
                pltpu.sync_copy(ei_hbm.at[1, pl.ds(base, SUP * C)], dstA)
                pltpu.async_copy(
                    x_hbm.at[srcA.at[pl.ds(sid * C, C)]], bufA, semA).wait()
                pltpu.sync_copy(
                    bufA, acc.at[dstA.at[pl.ds(sid * C, C)]], add=True)

        @pl.when(cid == 1)
        def _():
            def build(buf, ea_lo):
                for e in range(C):
                    buf[e, pl.ds(0, EA)] = eaall[ea_lo + e // EPC,
                                                 pl.ds((e % EPC) * EA, EA)]

            @pl.loop(0, NSUP)
            def _(s):
                row = cbase + s * SUP
                base = row * C
                pltpu.sync_copy(ei_hbm.at[1, pl.ds(base, SUP * C)], dstA)
                sd = [None] * SUP
                for j in range(SUP):
                    p = j % 3
                    if j % (SUP // 2) == 0:
                        pltpu.sync_copy(
                            ea8_hbm.at[pl.ds((row + j) * ER, SUP * ER // 2)],
                            eaall)
                    if j >= 3:
                        sd[j - 3].wait()
                    build(bufs[p], (j % (SUP // 2)) * ER)
                    sd[j] = pltpu.async_copy(
                        bufs[p], acc.at[dstA.at[pl.ds(j * C, C)]],
                        ssems[p], add=True)
                sd[SUP - 3].wait()
                sd[SUP - 2].wait()
                sd[SUP - 1].wait()

            @pl.when(sid < REM)
            def _():
                row = K0 * NS
                pltpu.sync_copy(ei_hbm.at[1, pl.ds(row * C, SUP * C)], dstA)
                pltpu.sync_copy(ea8_hbm.at[pl.ds((row + sid) * ER, ER)],
                                eaall.at[pl.ds(0, ER)])
                build(bufA, 0)
                pltpu.sync_copy(
                    bufA, acc.at[dstA.at[pl.ds(sid * C, C)]], add=True)

        plsc.subcore_barrier()

        def drain(out_hbm):
            def d(lo, n):
                pltpu.sync_copy(acc.at[pl.ds(lo, n)], bufA.at[pl.ds(0, n)])
                pltpu.sync_copy(bufA.at[pl.ds(0, n)], out_hbm.at[pl.ds(lo, n)])

            _striped(d, r0, RPT, C)

            @pl.when(last)
            def _():
                d(N - TAIL, TAIL)

        @pl.when(cid == 0)
        def _():
            drain(px_hbm)

        @pl.when(cid == 1)
        def _():
            drain(pm_hbm)

    return sc


def _tc_body(px, pm, x, wx, wself, we, bx, bself, be, out):
    gx = px[...]
    ga = pm[:, 0:EA]
    deg = pm[:, EA:EA + 1]
    summed = jnp.dot(gx, wx[...], preferred_element_type=jnp.float32)
    summed = summed + jnp.dot(ga, we[...], preferred_element_type=jnp.float32)
    summed = summed + deg * (bx[...] + be[...])
    agg = summed / jnp.maximum(deg, 1.0)
    out[...] = agg + jnp.dot(x[...], wself[...],
                             preferred_element_type=jnp.float32) + bself[...]


def kernel(x, edge_index, edge_attr, W_x, b_x, W_self, b_self, W_e, b_e):
    ei = edge_index.astype(jnp.int32)
    ea8 = edge_attr.reshape(E * EA // D, D)

    px, pm = _sc_aggregate()(x, ei, ea8)

    RB = 1000
    out = pl.pallas_call(
        _tc_body,
        grid=(N // RB,),
        in_specs=[
            pl.BlockSpec((RB, D), lambda i: (i, 0)),
            pl.BlockSpec((RB, D), lambda i: (i, 0)),
            pl.BlockSpec((RB, D), lambda i: (i, 0)),
            pl.BlockSpec((D, D), lambda i: (0, 0)),
            pl.BlockSpec((D, D), lambda i: (0, 0)),
            pl.BlockSpec((EA, D), lambda i: (0, 0)),
            pl.BlockSpec((1, D), lambda i: (0, 0)),
            pl.BlockSpec((1, D), lambda i: (0, 0)),
            pl.BlockSpec((1, D), lambda i: (0, 0)),
        ],
        out_specs=pl.BlockSpec((RB, D), lambda i: (i, 0)),
        out_shape=jax.ShapeDtypeStruct((N, D), jnp.float32),
    )(px, pm, x, W_x, W_self, W_e,
      b_x.reshape(1, D), b_self.reshape(1, D), b_e.reshape(1, D))
    return out


# speedup vs baseline: 6.2180x; 1.0071x over previous
import functools

import jax
import jax.numpy as jnp
from jax import lax
from jax.experimental import pallas as pl
from jax.experimental.pallas import tpu as pltpu
from jax.experimental.pallas import tpu_sc as plsc

N = 10000
E = 320000
D = 128
EA = 16
NC = 2
NS = 16
C = 64
SUP = 8
NCHUNK = E // C
K0 = NCHUNK // NS
NSUP = K0 // SUP
REM = NCHUNK - K0 * NS
RPT = 624
TAIL = N - NS * RPT
L = 16
ER = C * EA // D
EPC = D // EA


def _zero_vmem(ref, rows, width):
    z = jnp.zeros((L,), jnp.float32)

    @pl.loop(0, rows)
    def _(r):
        for cc in range(width // L):
            ref[r, pl.ds(cc * L, L)] = z


def _striped(copy, r0, total, chunk):
    done = 0
    while done < total:
        n = min(chunk, total - done)
        copy(r0 + done, n)
        done += n


def _sc_aggregate():
    mesh = plsc.VectorSubcoreMesh(
        core_axis_name="c", subcore_axis_name="s",
        num_cores=NC, num_subcores=NS)

    @functools.partial(
        pl.kernel,
        out_type=(
            jax.ShapeDtypeStruct((N, D), jnp.float32),
            jax.ShapeDtypeStruct((N, D), jnp.float32),
        ),
        mesh=mesh,
        scratch_types=[
            pltpu.VMEM_SHARED((N, D), jnp.float32),
            pltpu.VMEM((C, D), jnp.float32),
            pltpu.VMEM((C, D), jnp.float32),
            pltpu.VMEM((C, D), jnp.float32),
            pltpu.VMEM((SUP * C,), jnp.int32),
            pltpu.VMEM((SUP * C,), jnp.int32),
            pltpu.VMEM((SUP * C,), jnp.int32),
            pltpu.VMEM((SUP * C,), jnp.int32),
            pltpu.VMEM((SUP * ER // 2, D), jnp.float32),
            pltpu.SemaphoreType.DMA,
            pltpu.SemaphoreType.DMA,
            pltpu.SemaphoreType.DMA,
            pltpu.SemaphoreType.DMA,
            pltpu.SemaphoreType.DMA,
            pltpu.SemaphoreType.DMA,
            pltpu.SemaphoreType.DMA,
            pltpu.SemaphoreType.DMA,
        ],
    )
    def sc(x_hbm, ei_hbm, ea8_hbm,
           px_hbm, pm_hbm,
           acc, bufA, bufB, bufC, srcA, dstA, srcB, dstB, eaall,
           semA, semB, semC, semSA, semSB, semSC, semIA, semIB):
        cid = lax.axis_index("c")
        sid = lax.axis_index("s")
        r0 = sid * RPT
        last = sid == NS - 1
        bufs = (bufA, bufB, bufC)
        gsems = (semA, semB, semC)
        ssems = (semSA, semSB, semSC)
        cbase = sid * K0

        pieces = []
        done = 0
        while done < RPT:
            n = min(C, RPT - done)
            pieces.append((done, n))
            done += n

        _zero_vmem(bufA, C, D)
        zd = [pltpu.async_copy(bufA.at[pl.ds(0, n)],
                               acc.at[pl.ds(r0 + lo, n)], semA)
              for lo, n in pieces]

        @pl.when(last)
        def _():
            pltpu.sync_copy(bufA.at[pl.ds(0, TAIL)],
                            acc.at[pl.ds(N - TAIL, TAIL)])

        _zero_vmem(bufB, C, D)
        _zero_vmem(bufC, C, D)
        for d in zd:
            d.wait()
        ones = jnp.ones((L,), jnp.float32)

        @pl.loop(0, C)
        def _(r):
            bufA[r, pl.ds(EA, L)] = ones
            bufB[r, pl.ds(EA, L)] = ones
            bufC[r, pl.ds(EA, L)] = ones

        plsc.subcore_barrier()

        @pl.when(cid == 0)
        def _():
            def prefetch(s, srcall, dstall, sem):
                b = (cbase + s * SUP) * C
                d1 = pltpu.async_copy(
                    ei_hbm.at[0, pl.ds(b, SUP * C)], srcall, sem)
                d2 = pltpu.async_copy(
                    ei_hbm.at[1, pl.ds(b, SUP * C)], dstall, sem)
                return d1, d2

            def run_super(srcall, dstall):
                gd = [None] * SUP
                sd = [None] * SUP
                gd[0] = pltpu.async_copy(
                    x_hbm.at[srcall.at[pl.ds(0, C)]], bufs[0], gsems[0])
                gd[1] = pltpu.async_copy(
                    x_hbm.at[srcall.at[pl.ds(C, C)]], bufs[1], gsems[1])
                for j in range(SUP):
                    p = j % 3
                    gd[j].wait()
                    if j + 2 < SUP:
                        if j >= 1:
                            sd[j - 1].wait()
                        gd[j + 2] = pltpu.async_copy(
                            x_hbm.at[srcall.at[pl.ds((j + 2) * C, C)]],
                            bufs[(j + 2) % 3], gsems[(j + 2) % 3])
                    sd[j] = pltpu.async_copy(
                        bufs[p], acc.at[dstall.at[pl.ds(j * C, C)]],
                        ssems[p], add=True)
                sd[SUP - 3].wait()
                sd[SUP - 2].wait()
                sd[SUP - 1].wait()

            b0 = cbase * C
            pltpu.sync_copy(ei_hbm.at[0, pl.ds(b0, SUP * C)], srcA)
            pltpu.sync_copy(ei_hbm.at[1, pl.ds(b0, SUP * C)], dstA)

            @pl.loop(0, (NSUP - 1) // 2)
            def _(t):
                pfB = prefetch(2 * t + 1, srcB, dstB, semIB)
                run_super(srcA, dstA)
                pfB[0].wait()
                pfB[1].wait()
                pfA = prefetch(2 * t + 2, srcA, dstA, semIA)
                run_super(srcB, dstB)
                pfA[0].wait()
                pfA[1].wait()

            run_super(srcA, dstA)

            @pl.when(sid < REM)
            def _():
                base = K0 * NS * C
                pltpu.sync_copy(ei_hbm.at[0, pl.ds(base, SUP * C)], srcA)
                pltpu.sync_copy(ei_hbm.at[1, pl.ds(base, SUP * C)], dstA)
                pltpu.async_copy(
                    x_hbm.at[srcA.at[pl.ds(sid * C, C)]], bufA, semA).wait()
                pltpu.sync_copy(
                    bufA, acc.at[dstA.at[pl.ds(sid * C, C)]], add=True)

        @pl.when(cid == 1)
        def _():
            def build(buf, ea_lo):
                for e in range(C):
                    buf[e, pl.ds(0, EA)] = eaall[ea_lo + e // EPC,
                                                 pl.ds((e % EPC) * EA, EA)]

            @pl.loop(0, NSUP)
            def _(s):
                row = cbase + s * SUP
                base = row * C
                pltpu.sync_copy(ei_hbm.at[1, pl.ds(base, SUP * C)], dstA)
                sd = [None] * SUP
                for j in range(SUP):
                    p = j % 3
                    if j % (SUP // 2) == 0:
                        pltpu.sync_copy(
                            ea8_hbm.at[pl.ds((row + j) * ER, SUP * ER // 2)],
                            eaall)
                    if j >= 3:
                        sd[j - 3].wait()
                    build(bufs[p], (j % (SUP // 2)) * ER)
                    sd[j] = pltpu.async_copy(
                        bufs[p], acc.at[dstA.at[pl.ds(j * C, C)]],
                        ssems[p], add=True)
                sd[SUP - 3].wait()
                sd[SUP - 2].wait()
                sd[SUP - 1].wait()

            @pl.when(sid < REM)
            def _():
                row = K0 * NS
                pltpu.sync_copy(ei_hbm.at[1, pl.ds(row * C, SUP * C)], dstA)
                pltpu.sync_copy(ea8_hbm.at[pl.ds((row + sid) * ER, ER)],
                                eaall.at[pl.ds(0, ER)])
                build(bufA, 0)
                pltpu.sync_copy(
                    bufA, acc.at[dstA.at[pl.ds(sid * C, C)]], add=True)

        plsc.subcore_barrier()

        def drain(out_hbm):
            sd = [None, None, None]
            for i, (lo, n) in enumerate(pieces):
                p = i % 3
                if sd[p] is not None:
                    sd[p].wait()
                pltpu.sync_copy(acc.at[pl.ds(r0 + lo, n)],
                                bufs[p].at[pl.ds(0, n)])
                sd[p] = pltpu.async_copy(bufs[p].at[pl.ds(0, n)],
                                         out_hbm.at[pl.ds(r0 + lo, n)],
                                         ssems[p])
            for p in range(3):
                if sd[p] is not None:
                    sd[p].wait()

            @pl.when(last)
            def _():
                pltpu.sync_copy(acc.at[pl.ds(N - TAIL, TAIL)],
                                bufA.at[pl.ds(0, TAIL)])
                pltpu.sync_copy(bufA.at[pl.ds(0, TAIL)],
                                out_hbm.at[pl.ds(N - TAIL, TAIL)])

        @pl.when(cid == 0)
        def _():
            drain(px_hbm)

        @pl.when(cid == 1)
        def _():
            drain(pm_hbm)

    return sc


def _tc_body(px, pm, x, wx, wself, we, bx, bself, be, out):
    gx = px[...]
    ga = pm[:, 0:EA]
    deg = pm[:, EA:EA + 1]
    summed = jnp.dot(gx, wx[...], preferred_element_type=jnp.float32)
    summed = summed + jnp.dot(ga, we[...], preferred_element_type=jnp.float32)
    summed = summed + deg * (bx[...] + be[...])
    agg = summed / jnp.maximum(deg, 1.0)
    out[...] = agg + jnp.dot(x[...], wself[...],
                             preferred_element_type=jnp.float32) + bself[...]


def kernel(x, edge_index, edge_attr, W_x, b_x, W_self, b_self, W_e, b_e):
    ei = edge_index.astype(jnp.int32)
    ea8 = edge_attr.reshape(E * EA // D, D)

    px, pm = _sc_aggregate()(x, ei, ea8)

    RB = 1000
    out = pl.pallas_call(
        _tc_body,
        grid=(N // RB,),
        in_specs=[
            pl.BlockSpec((RB, D), lambda i: (i, 0)),
            pl.BlockSpec((RB, D), lambda i: (i, 0)),
            pl.BlockSpec((RB, D), lambda i: (i, 0)),
            pl.BlockSpec((D, D), lambda i: (0, 0)),
            pl.BlockSpec((D, D), lambda i: (0, 0)),
            pl.BlockSpec((EA, D), lambda i: (0, 0)),
            pl.BlockSpec((1, D), lambda i: (0, 0)),
            pl.BlockSpec((1, D), lambda i: (0, 0)),
            pl.BlockSpec((1, D), lambda i: (0, 0)),
        ],
        out_specs=pl.BlockSpec((RB, D), lambda i: (i, 0)),
        out_shape=jax.ShapeDtypeStruct((N, D), jnp.float32),
    )(px, pm, x, W_x, W_self, W_e,
      b_x.reshape(1, D), b_self.reshape(1, D), b_e.reshape(1, D))
    return out
